# baseline (speedup 1.0000x reference)
import functools

import jax
import jax.numpy as jnp
from jax import lax
from jax.experimental import pallas as pl
from jax.experimental.pallas import tpu as pltpu
from jax.experimental.pallas import tpu_sc as plsc

N = 10000
E = 320000
D = 128
N_PAD = 10240
E_PAD = 327680
NT = 16
NSC = 2
CHUNK = 128
TILE_E = E_PAD // (NSC * NT)
NCHUNK = TILE_E // CHUNK
NPIPE = NCHUNK // 2
ZROWS = 32
ROWS_T = N_PAD // NT
BN = 1024
GRID = N_PAD // BN


def _head_outputs(hcur, W_ref, al_ref, ar_ref, z_ref, elt_ref, ert_ref, Hn):
  for h in range(Hn):
    zh = jnp.dot(hcur, W_ref[h], preferred_element_type=jnp.float32)
    z_ref[h] = zh
    elt_ref[:, h:h + 1] = jnp.sum(zh * al_ref[h][None, :], axis=1,
                                  keepdims=True)
    ert_ref[:, h:h + 1] = jnp.sum(zh * ar_ref[h][None, :], axis=1,
                                  keepdims=True)


def _make_tc_first(H):
  def body(x_ref, W_ref, al_ref, ar_ref, z_ref, elt_ref, ert_ref):
    _head_outputs(x_ref[...], W_ref, al_ref, ar_ref, z_ref, elt_ref, ert_ref,
                  H)

  return pl.pallas_call(
      body,
      grid=(GRID,),
      in_specs=[
          pl.BlockSpec((BN, D), lambda i: (i, 0)),
          pl.BlockSpec((H, D, D), lambda i: (0, 0, 0)),
          pl.BlockSpec((H, D), lambda i: (0, 0)),
          pl.BlockSpec((H, D), lambda i: (0, 0)),
      ],
      out_specs=[
          pl.BlockSpec((H, BN, D), lambda i: (0, i, 0)),
          pl.BlockSpec((BN, H), lambda i: (i, 0)),
          pl.BlockSpec((BN, H), lambda i: (i, 0)),
      ],
      out_shape=[
          jax.ShapeDtypeStruct((H, N_PAD, D), jnp.float32),
          jax.ShapeDtypeStruct((N_PAD, H), jnp.float32),
          jax.ShapeDtypeStruct((N_PAD, H), jnp.float32),
      ],
  )


def _aggregate(acc_ref, st_ref, Hp):
  tot = None
  for h in range(Hp):
    zc = acc_ref[h, 0] + acc_ref[h, 1]
    sv = st_ref[:, 2 * h:2 * h + 1] + st_ref[:, 2 * h + 1:2 * h + 2]
    v = zc / (sv + 1e-9)
    tot = v if tot is None else tot + v
  a = tot * (1.0 / Hp)
  return jnp.where(a > 0, a, jnp.exp(jnp.minimum(a, 0.0)) - 1.0)


def _gru(a, hp, gwx_ref, gwh_ref, gbx_ref, gbh_ref):
  gx = jnp.dot(a, gwx_ref[...], preferred_element_type=jnp.float32)
  gx = gx + gbx_ref[...][None, :]
  gh = jnp.dot(hp, gwh_ref[...], preferred_element_type=jnp.float32)
  gh = gh + gbh_ref[...][None, :]
  r = jax.nn.sigmoid(gx[:, :D] + gh[:, :D])
  zt = jax.nn.sigmoid(gx[:, D:2 * D] + gh[:, D:2 * D])
  ng = jnp.tanh(gx[:, 2 * D:] + r * gh[:, 2 * D:])
  return (1.0 - zt) * ng + zt * hp


def _make_tc_trans(Hp, Hn, has_gru):
  if has_gru:
    def body(acc_ref, st_ref, hprev_ref, gwx_ref, gwh_ref, gbx_ref, gbh_ref,
             W_ref, al_ref, ar_ref, h_ref, z_ref, elt_ref, ert_ref):
      a = _aggregate(acc_ref, st_ref, Hp)
      hcur = _gru(a, hprev_ref[...], gwx_ref, gwh_ref, gbx_ref, gbh_ref)
      h_ref[...] = hcur
      _head_outputs(hcur, W_ref, al_ref, ar_ref, z_ref, elt_ref, ert_ref, Hn)

    extra_specs = [
        pl.BlockSpec((BN, D), lambda i: (i, 0)),
        pl.BlockSpec((D, 3 * D), lambda i: (0, 0)),
        pl.BlockSpec((D, 3 * D), lambda i: (0, 0)),
        pl.BlockSpec((3 * D,), lambda i: (0,)),
        pl.BlockSpec((3 * D,), lambda i: (0,)),
    ]
  else:
    def body(acc_ref, st_ref, W_ref, al_ref, ar_ref, h_ref, z_ref, elt_ref,
             ert_ref):
      a = _aggregate(acc_ref, st_ref, Hp)
      h_ref[...] = a
      _head_outputs(a, W_ref, al_ref, ar_ref, z_ref, elt_ref, ert_ref, Hn)

    extra_specs = []

  return pl.pallas_call(
      body,
      grid=(GRID,),
      in_specs=[
          pl.BlockSpec((Hp, NSC, BN, D), lambda i: (0, 0, i, 0)),
          pl.BlockSpec((BN, NSC * Hp), lambda i: (i, 0)),
      ] + extra_specs + [
          pl.BlockSpec((Hn, D, D), lambda i: (0, 0, 0)),
          pl.BlockSpec((Hn, D), lambda i: (0, 0)),
          pl.BlockSpec((Hn, D), lambda i: (0, 0)),
      ],
      out_specs=[
          pl.BlockSpec((BN, D), lambda i: (i, 0)),
          pl.BlockSpec((Hn, BN, D), lambda i: (0, i, 0)),
          pl.BlockSpec((BN, Hn), lambda i: (i, 0)),
          pl.BlockSpec((BN, Hn), lambda i: (i, 0)),
      ],
      out_shape=[
          jax.ShapeDtypeStruct((N_PAD, D), jnp.float32),
          jax.ShapeDtypeStruct((Hn, N_PAD, D), jnp.float32),
          jax.ShapeDtypeStruct((N_PAD, Hn), jnp.float32),
          jax.ShapeDtypeStruct((N_PAD, Hn), jnp.float32),
      ],
  )


def _make_tc_final():
  def body(acc_ref, st_ref, hprev_ref, gwx_ref, gwh_ref, gbx_ref, gbh_ref,
           W5_ref, b5_ref, out_ref):
    a = _aggregate(acc_ref, st_ref, 1)
    hcur = _gru(a, hprev_ref[...], gwx_ref, gwh_ref, gbx_ref, gbh_ref)
    y = jnp.dot(hcur, W5_ref[...], preferred_element_type=jnp.float32)
    out_ref[...] = jax.nn.sigmoid(y + b5_ref[...][None, :])

  return pl.pallas_call(
      body,
      grid=(GRID,),
      in_specs=[
          pl.BlockSpec((1, NSC, BN, D), lambda i: (0, 0, i, 0)),
          pl.BlockSpec((BN, NSC), lambda i: (i, 0)),
          pl.BlockSpec((BN, D), lambda i: (i, 0)),
          pl.BlockSpec((D, 3 * D), lambda i: (0, 0)),
          pl.BlockSpec((D, 3 * D), lambda i: (0, 0)),
          pl.BlockSpec((3 * D,), lambda i: (0,)),
          pl.BlockSpec((3 * D,), lambda i: (0,)),
          pl.BlockSpec((D, D), lambda i: (0, 0)),
          pl.BlockSpec((D,), lambda i: (0,)),
      ],
      out_specs=[pl.BlockSpec((BN, D), lambda i: (i, 0))],
      out_shape=[jax.ShapeDtypeStruct((N_PAD, D), jnp.float32)],
  )


def _make_sc_edge(H):
  mesh = plsc.VectorSubcoreMesh(core_axis_name="c", subcore_axis_name="s",
                                num_cores=NSC, num_subcores=NT)

  out_type = (
      jax.ShapeDtypeStruct((H, NSC, N_PAD, D), jnp.float32),
      jax.ShapeDtypeStruct((H, NSC, N_PAD), jnp.float32),
  )
  scratch = [
      pltpu.VMEM_SHARED((N_PAD, D), jnp.float32),
      pltpu.VMEM_SHARED((N_PAD,), jnp.float32),
      pltpu.VMEM_SHARED((N_PAD,), jnp.float32),
      pltpu.VMEM_SHARED((N_PAD,), jnp.float32),
      [pltpu.VMEM((CHUNK,), jnp.int32) for _ in range(2)],
      [pltpu.VMEM((CHUNK,), jnp.int32) for _ in range(2)],
      [pltpu.VMEM((CHUNK,), jnp.float32) for _ in range(2)],
      [pltpu.VMEM((CHUNK,), jnp.float32) for _ in range(2)],
      [pltpu.VMEM((CHUNK,), jnp.float32) for _ in range(2)],
      [pltpu.VMEM((CHUNK, D), jnp.float32) for _ in range(2)],
      pltpu.VMEM((ZROWS, D), jnp.float32),
      pltpu.VMEM((ROWS_T,), jnp.float32),
      [pltpu.SemaphoreType.DMA for _ in range(2)],
      [pltpu.SemaphoreType.DMA for _ in range(2)],
      [pltpu.SemaphoreType.DMA for _ in range(2)],
      [pltpu.SemaphoreType.DMA for _ in range(2)],
  ]

  @functools.partial(pl.kernel, out_type=out_type, mesh=mesh,
                     scratch_types=scratch)
  def k(*refs):
    (z_hbm, els, ers, src_hbm, dst_hbm, acc_out, s_out,
     acc_sp, s_sp, el_sp, er_sp, srcv, dstv, elv, erv, exv, rows, zbuf, zsv,
     gsem, ssem, esem, xsem) = (refs[0], refs[1:1 + H], refs[1 + H:1 + 2 * H],
                                *refs[1 + 2 * H:])
    c = lax.axis_index("c")
    t = lax.axis_index("s")
    rbase = t * ROWS_T
    ebase = c * (E_PAD // NSC) + t * TILE_E

    def zrow(i, carry):
      for q in range(D // 16):
        zbuf[i, pl.ds(q * 16, 16)] = jnp.zeros((16,), jnp.float32)
      return carry

    lax.fori_loop(0, ZROWS, zrow, 0)

    def zvec(i, carry):
      zsv[pl.ds(i * 16, 16)] = jnp.zeros((16,), jnp.float32)
      return carry

    lax.fori_loop(0, ROWS_T // 16, zvec, 0)

    for h in range(H):

      def load_idx(b, base):
        pltpu.sync_copy(src_hbm.at[pl.ds(base, CHUNK)], srcv[b])
        pltpu.sync_copy(dst_hbm.at[pl.ds(base, CHUNK)], dstv[b])
        pltpu.async_copy(el_sp.at[srcv[b]], elv[b], esem[b])
        pltpu.async_copy(er_sp.at[dstv[b]], erv[b], esem[b])
        pltpu.async_copy(z_hbm.at[h].at[srcv[b]], rows[b], gsem[b])

      def process(b):
        pltpu.make_async_copy(el_sp.at[srcv[b]], elv[b], esem[b]).wait()
        pltpu.make_async_copy(er_sp.at[dstv[b]], erv[b], esem[b]).wait()
        for q in range(CHUNK // 16):
          sl = pl.ds(q * 16, 16)
          e = elv[b][sl] + erv[b][sl]
          e = jnp.where(e >= 0.0, e, 0.2 * e)
          exv[b][sl] = jnp.exp(e)
        pltpu.async_copy(exv[b], s_sp.at[dstv[b]], xsem[b], add=True)
        pltpu.make_async_copy(z_hbm.at[h].at[srcv[b]], rows[b],
                              gsem[b]).wait()

        def row_body(g, rcarry):
          exvec = exv[b][pl.ds(g * 16, 16)]
          for r in range(16):
            i = g * 16 + r
            scv = jnp.full((16,), exvec[r], jnp.float32)
            for q in range(D // 16):
              sl = pl.ds(q * 16, 16)
              rows[b][i, sl] = rows[b][i, sl] * scv
          return rcarry

        lax.fori_loop(0, CHUNK // 16, row_body, 0)
        pltpu.async_copy(rows[b], acc_sp.at[dstv[b]], ssem[b], add=True)

      def drain_scatter(b):
        pltpu.make_async_copy(rows[b], acc_sp.at[dstv[b]], ssem[b]).wait()
        pltpu.make_async_copy(exv[b], s_sp.at[dstv[b]], xsem[b]).wait()

      pltpu.sync_copy(els[h].at[pl.ds(rbase, ROWS_T)],
                      el_sp.at[pl.ds(rbase, ROWS_T)])
      pltpu.sync_copy(ers[h].at[pl.ds(rbase, ROWS_T)],
                      er_sp.at[pl.ds(rbase, ROWS_T)])
      for jj in range(ROWS_T // ZROWS):
        pltpu.sync_copy(zbuf, acc_sp.at[pl.ds(rbase + jj * ZROWS, ZROWS)])
      pltpu.sync_copy(zsv, s_sp.at[pl.ds(rbase, ROWS_T)])
      plsc.subcore_barrier()

      load_idx(0, ebase)

      def pipe_body(jj, carry):
        base_a = ebase + jj * (2 * CHUNK)

        @pl.when(jj > 0)
        def _():
          drain_scatter(1)

        load_idx(1, base_a + CHUNK)
        process(0)

        @pl.when(jj < NPIPE - 1)
        def _():
          drain_scatter(0)
          load_idx(0, base_a + 2 * CHUNK)

        process(1)
        return carry

      lax.fori_loop(0, NPIPE, pipe_body, 0)
      drain_scatter(0)
      drain_scatter(1)
      plsc.subcore_barrier()

      pltpu.sync_copy(acc_sp.at[pl.ds(rbase, ROWS_T)],
                      acc_out.at[h, c, pl.ds(rbase, ROWS_T)])
      pltpu.sync_copy(s_sp.at[pl.ds(rbase, ROWS_T)],
                      s_out.at[h, c, pl.ds(rbase, ROWS_T)])

  return k


_tc_first = _make_tc_first(4)
_tc_t2 = _make_tc_trans(4, 4, False)
_tc_t3 = _make_tc_trans(4, 4, True)
_tc_t4 = _make_tc_trans(4, 1, True)
_tc_final = _make_tc_final()
_sc_edge4 = _make_sc_edge(4)
_sc_edge1 = _make_sc_edge(1)


def _st(s):
  return jnp.transpose(s, (2, 0, 1)).reshape(N_PAD, -1)


def kernel(x, edge_index, W1, al1, ar1, W2, al2, ar2, W3, al3, ar3,
           W4, al4, ar4, gru_Wx, gru_Wh, gru_bx, gru_bh, W5, b5):
  x_pad = jnp.pad(x, ((0, N_PAD - N), (0, 0)))
  src = edge_index[0]
  dst = edge_index[1]
  pad_idx = N + (jnp.arange(E_PAD - E, dtype=jnp.int32) % (N_PAD - N))
  srcp = jnp.concatenate([src, pad_idx])
  dstp = jnp.concatenate([dst, pad_idx])
  gru = (gru_Wx, gru_Wh, gru_bx, gru_bh)
  W5p = jnp.pad(W5, ((0, 0), (0, D - 1)))
  b5p = jnp.pad(b5, (0, D - 1))

  def sc(fn, z, elt, ert, H):
    cols = lambda m: [m[:, h] for h in range(H)]
    return fn(z, *cols(elt), *cols(ert), srcp, dstp)

  z, elt, ert = _tc_first(x_pad, W1, al1, ar1)
  acc, s = sc(_sc_edge4, z, elt, ert, 4)
  h1, z, elt, ert = _tc_t2(acc, _st(s), W2, al2, ar2)
  acc, s = sc(_sc_edge4, z, elt, ert, 4)
  h2, z, elt, ert = _tc_t3(acc, _st(s), h1, *gru, W3, al3, ar3)
  acc, s = sc(_sc_edge4, z, elt, ert, 4)
  h3, z, elt, ert = _tc_t4(acc, _st(s), h2, *gru, W4, al4, ar4)
  acc, s = sc(_sc_edge1, z, elt, ert, 1)
  (out_full,) = _tc_final(acc, _st(s), h3, *gru, W5p, b5p)
  return out_full[:N, :1]


# speedup vs baseline: 1.1794x; 1.1794x over previous
import functools

import jax
import jax.numpy as jnp
from jax import lax
from jax.experimental import pallas as pl
from jax.experimental.pallas import tpu as pltpu
from jax.experimental.pallas import tpu_sc as plsc

N = 10000
E = 320000
D = 128
N_PAD = 10240
E_PAD = 327680
NT = 16
NSC = 2
CHUNK = 128
TILE_E = E_PAD // (NSC * NT)
NCHUNK = TILE_E // CHUNK
NPIPE = NCHUNK // 2
ZROWS = 32
ROWS_T = N_PAD // NT
BN = 1024
GRID = N_PAD // BN


def _head_outputs(hcur, W_ref, al_ref, ar_ref, z_ref, elt_ref, ert_ref, Hn):
  for h in range(Hn):
    zh = jnp.dot(hcur, W_ref[h], preferred_element_type=jnp.float32)
    z_ref[h] = zh
    elt_ref[:, h:h + 1] = jnp.sum(zh * al_ref[h][None, :], axis=1,
                                  keepdims=True)
    ert_ref[:, h:h + 1] = jnp.sum(zh * ar_ref[h][None, :], axis=1,
                                  keepdims=True)


def _make_tc_first(H):
  def body(x_ref, W_ref, al_ref, ar_ref, z_ref, elt_ref, ert_ref):
    _head_outputs(x_ref[...], W_ref, al_ref, ar_ref, z_ref, elt_ref, ert_ref,
                  H)

  return pl.pallas_call(
      body,
      grid=(GRID,),
      in_specs=[
          pl.BlockSpec((BN, D), lambda i: (i, 0)),
          pl.BlockSpec((H, D, D), lambda i: (0, 0, 0)),
          pl.BlockSpec((H, D), lambda i: (0, 0)),
          pl.BlockSpec((H, D), lambda i: (0, 0)),
      ],
      out_specs=[
          pl.BlockSpec((H, BN, D), lambda i: (0, i, 0)),
          pl.BlockSpec((BN, H), lambda i: (i, 0)),
          pl.BlockSpec((BN, H), lambda i: (i, 0)),
      ],
      out_shape=[
          jax.ShapeDtypeStruct((H, N_PAD, D), jnp.float32),
          jax.ShapeDtypeStruct((N_PAD, H), jnp.float32),
          jax.ShapeDtypeStruct((N_PAD, H), jnp.float32),
      ],
  )


def _aggregate(acc_ref, st_ref, Hp):
  tot = None
  for h in range(Hp):
    zc = acc_ref[h, 0] + acc_ref[h, 1]
    sv = st_ref[:, 2 * h:2 * h + 1] + st_ref[:, 2 * h + 1:2 * h + 2]
    v = zc / (sv + 1e-9)
    tot = v if tot is None else tot + v
  a = tot * (1.0 / Hp)
  return jnp.where(a > 0, a, jnp.exp(jnp.minimum(a, 0.0)) - 1.0)


def _gru(a, hp, gwx_ref, gwh_ref, gbx_ref, gbh_ref):
  gx = jnp.dot(a, gwx_ref[...], preferred_element_type=jnp.float32)
  gx = gx + gbx_ref[...][None, :]
  gh = jnp.dot(hp, gwh_ref[...], preferred_element_type=jnp.float32)
  gh = gh + gbh_ref[...][None, :]
  r = jax.nn.sigmoid(gx[:, :D] + gh[:, :D])
  zt = jax.nn.sigmoid(gx[:, D:2 * D] + gh[:, D:2 * D])
  ng = jnp.tanh(gx[:, 2 * D:] + r * gh[:, 2 * D:])
  return (1.0 - zt) * ng + zt * hp


def _make_tc_trans(Hp, Hn, has_gru):
  if has_gru:
    def body(acc_ref, st_ref, hprev_ref, gwx_ref, gwh_ref, gbx_ref, gbh_ref,
             W_ref, al_ref, ar_ref, h_ref, z_ref, elt_ref, ert_ref):
      a = _aggregate(acc_ref, st_ref, Hp)
      hcur = _gru(a, hprev_ref[...], gwx_ref, gwh_ref, gbx_ref, gbh_ref)
      h_ref[...] = hcur
      _head_outputs(hcur, W_ref, al_ref, ar_ref, z_ref, elt_ref, ert_ref, Hn)

    extra_specs = [
        pl.BlockSpec((BN, D), lambda i: (i, 0)),
        pl.BlockSpec((D, 3 * D), lambda i: (0, 0)),
        pl.BlockSpec((D, 3 * D), lambda i: (0, 0)),
        pl.BlockSpec((3 * D,), lambda i: (0,)),
        pl.BlockSpec((3 * D,), lambda i: (0,)),
    ]
  else:
    def body(acc_ref, st_ref, W_ref, al_ref, ar_ref, h_ref, z_ref, elt_ref,
             ert_ref):
      a = _aggregate(acc_ref, st_ref, Hp)
      h_ref[...] = a
      _head_outputs(a, W_ref, al_ref, ar_ref, z_ref, elt_ref, ert_ref, Hn)

    extra_specs = []

  return pl.pallas_call(
      body,
      grid=(GRID,),
      in_specs=[
          pl.BlockSpec((Hp, NSC, BN, D), lambda i: (0, 0, i, 0)),
          pl.BlockSpec((BN, NSC * Hp), lambda i: (i, 0)),
      ] + extra_specs + [
          pl.BlockSpec((Hn, D, D), lambda i: (0, 0, 0)),
          pl.BlockSpec((Hn, D), lambda i: (0, 0)),
          pl.BlockSpec((Hn, D), lambda i: (0, 0)),
      ],
      out_specs=[
          pl.BlockSpec((BN, D), lambda i: (i, 0)),
          pl.BlockSpec((Hn, BN, D), lambda i: (0, i, 0)),
          pl.BlockSpec((BN, Hn), lambda i: (i, 0)),
          pl.BlockSpec((BN, Hn), lambda i: (i, 0)),
      ],
      out_shape=[
          jax.ShapeDtypeStruct((N_PAD, D), jnp.float32),
          jax.ShapeDtypeStruct((Hn, N_PAD, D), jnp.float32),
          jax.ShapeDtypeStruct((N_PAD, Hn), jnp.float32),
          jax.ShapeDtypeStruct((N_PAD, Hn), jnp.float32),
      ],
  )


def _make_tc_final():
  def body(acc_ref, st_ref, hprev_ref, gwx_ref, gwh_ref, gbx_ref, gbh_ref,
           W5_ref, b5_ref, out_ref):
    a = _aggregate(acc_ref, st_ref, 1)
    hcur = _gru(a, hprev_ref[...], gwx_ref, gwh_ref, gbx_ref, gbh_ref)
    y = jnp.dot(hcur, W5_ref[...], preferred_element_type=jnp.float32)
    out_ref[...] = jax.nn.sigmoid(y + b5_ref[...][None, :])

  return pl.pallas_call(
      body,
      grid=(GRID,),
      in_specs=[
          pl.BlockSpec((1, NSC, BN, D), lambda i: (0, 0, i, 0)),
          pl.BlockSpec((BN, NSC), lambda i: (i, 0)),
          pl.BlockSpec((BN, D), lambda i: (i, 0)),
          pl.BlockSpec((D, 3 * D), lambda i: (0, 0)),
          pl.BlockSpec((D, 3 * D), lambda i: (0, 0)),
          pl.BlockSpec((3 * D,), lambda i: (0,)),
          pl.BlockSpec((3 * D,), lambda i: (0,)),
          pl.BlockSpec((D, D), lambda i: (0, 0)),
          pl.BlockSpec((D,), lambda i: (0,)),
      ],
      out_specs=[pl.BlockSpec((BN, D), lambda i: (i, 0))],
      out_shape=[jax.ShapeDtypeStruct((N_PAD, D), jnp.float32)],
  )


def _make_sc_edge(H):
  mesh = plsc.VectorSubcoreMesh(core_axis_name="c", subcore_axis_name="s",
                                num_cores=NSC, num_subcores=NT)

  out_type = (
      jax.ShapeDtypeStruct((H, NSC, N_PAD, D), jnp.float32),
      jax.ShapeDtypeStruct((H, NSC, N_PAD), jnp.float32),
  )
  scratch = [
      pltpu.VMEM_SHARED((N_PAD, D), jnp.float32),
      pltpu.VMEM_SHARED((N_PAD,), jnp.float32),
      pltpu.VMEM_SHARED((N_PAD,), jnp.float32),
      pltpu.VMEM_SHARED((N_PAD,), jnp.float32),
      [pltpu.VMEM((CHUNK,), jnp.int32) for _ in range(2)],
      [pltpu.VMEM((CHUNK,), jnp.int32) for _ in range(2)],
      [pltpu.VMEM((CHUNK,), jnp.float32) for _ in range(2)],
      [pltpu.VMEM((CHUNK,), jnp.float32) for _ in range(2)],
      [pltpu.VMEM((CHUNK,), jnp.float32) for _ in range(2)],
      [pltpu.VMEM((CHUNK, D), jnp.float32) for _ in range(2)],
      pltpu.VMEM((ZROWS, D), jnp.float32),
      pltpu.VMEM((ROWS_T,), jnp.float32),
      [pltpu.SemaphoreType.DMA for _ in range(2)],
      [pltpu.SemaphoreType.DMA for _ in range(2)],
      [pltpu.SemaphoreType.DMA for _ in range(2)],
      [pltpu.SemaphoreType.DMA for _ in range(2)],
      [pltpu.SemaphoreType.DMA for _ in range(2)],
  ]

  @functools.partial(pl.kernel, out_type=out_type, mesh=mesh,
                     scratch_types=scratch)
  def k(*refs):
    (z_hbm, els, ers, src_hbm, dst_hbm, acc_out, s_out,
     acc_sp, s_sp, el_sp, er_sp, srcv, dstv, elv, erv, exv, rows, zbuf, zsv,
     gsem, ssem, esem, xsem, isem) = (refs[0], refs[1:1 + H],
                                      refs[1 + H:1 + 2 * H],
                                      *refs[1 + 2 * H:])
    c = lax.axis_index("c")
    t = lax.axis_index("s")
    rbase = t * ROWS_T
    ebase = c * (E_PAD // NSC) + t * TILE_E

    def zrow(i, carry):
      for q in range(D // 16):
        zbuf[i, pl.ds(q * 16, 16)] = jnp.zeros((16,), jnp.float32)
      return carry

    lax.fori_loop(0, ZROWS, zrow, 0)

    def zvec(i, carry):
      zsv[pl.ds(i * 16, 16)] = jnp.zeros((16,), jnp.float32)
      return carry

    lax.fori_loop(0, ROWS_T // 16, zvec, 0)

    for h in range(H):

      def load_idx(b, base):
        scp = pltpu.async_copy(src_hbm.at[pl.ds(base, CHUNK)], srcv[b],
                               isem[b])
        dcp = pltpu.async_copy(dst_hbm.at[pl.ds(base, CHUNK)], dstv[b],
                               isem[b])
        scp.wait()
        dcp.wait()
        pltpu.async_copy(el_sp.at[srcv[b]], elv[b], esem[b])
        pltpu.async_copy(er_sp.at[dstv[b]], erv[b], esem[b])
        pltpu.async_copy(z_hbm.at[h].at[srcv[b]], rows[b], gsem[b])

      def process(b):
        pltpu.make_async_copy(el_sp.at[srcv[b]], elv[b], esem[b]).wait()
        pltpu.make_async_copy(er_sp.at[dstv[b]], erv[b], esem[b]).wait()
        for q in range(CHUNK // 16):
          sl = pl.ds(q * 16, 16)
          e = elv[b][sl] + erv[b][sl]
          e = jnp.where(e >= 0.0, e, 0.2 * e)
          exv[b][sl] = jnp.exp(e)
        pltpu.async_copy(exv[b], s_sp.at[dstv[b]], xsem[b], add=True)
        pltpu.make_async_copy(z_hbm.at[h].at[srcv[b]], rows[b],
                              gsem[b]).wait()

        def row_body(g, rcarry):
          exvec = exv[b][pl.ds(g * 16, 16)]
          for r in range(16):
            i = g * 16 + r
            scv = jnp.full((16,), exvec[r], jnp.float32)
            for q in range(D // 16):
              sl = pl.ds(q * 16, 16)
              rows[b][i, sl] = rows[b][i, sl] * scv
          return rcarry

        lax.fori_loop(0, CHUNK // 16, row_body, 0)
        pltpu.async_copy(rows[b], acc_sp.at[dstv[b]], ssem[b], add=True)

      def drain_scatter(b):
        pltpu.make_async_copy(rows[b], acc_sp.at[dstv[b]], ssem[b]).wait()
        pltpu.make_async_copy(exv[b], s_sp.at[dstv[b]], xsem[b]).wait()

      pltpu.sync_copy(els[h].at[pl.ds(rbase, ROWS_T)],
                      el_sp.at[pl.ds(rbase, ROWS_T)])
      pltpu.sync_copy(ers[h].at[pl.ds(rbase, ROWS_T)],
                      er_sp.at[pl.ds(rbase, ROWS_T)])
      for jj in range(ROWS_T // ZROWS):
        pltpu.sync_copy(zbuf, acc_sp.at[pl.ds(rbase + jj * ZROWS, ZROWS)])
      pltpu.sync_copy(zsv, s_sp.at[pl.ds(rbase, ROWS_T)])
      plsc.subcore_barrier()

      load_idx(0, ebase)

      def pipe_body(jj, carry):
        base_a = ebase + jj * (2 * CHUNK)

        @pl.when(jj > 0)
        def _():
          drain_scatter(1)

        load_idx(1, base_a + CHUNK)
        process(0)

        @pl.when(jj < NPIPE - 1)
        def _():
          drain_scatter(0)
          load_idx(0, base_a + 2 * CHUNK)

        process(1)
        return carry

      lax.fori_loop(0, NPIPE, pipe_body, 0)
      drain_scatter(0)
      drain_scatter(1)
      plsc.subcore_barrier()

      pltpu.sync_copy(acc_sp.at[pl.ds(rbase, ROWS_T)],
                      acc_out.at[h, c, pl.ds(rbase, ROWS_T)])
      pltpu.sync_copy(s_sp.at[pl.ds(rbase, ROWS_T)],
                      s_out.at[h, c, pl.ds(rbase, ROWS_T)])

  return k


_tc_first = _make_tc_first(4)
_tc_t2 = _make_tc_trans(4, 4, False)
_tc_t3 = _make_tc_trans(4, 4, True)
_tc_t4 = _make_tc_trans(4, 1, True)
_tc_final = _make_tc_final()
_sc_edge4 = _make_sc_edge(4)
_sc_edge1 = _make_sc_edge(1)


def _st(s):
  return jnp.transpose(s, (2, 0, 1)).reshape(N_PAD, -1)


def kernel(x, edge_index, W1, al1, ar1, W2, al2, ar2, W3, al3, ar3,
           W4, al4, ar4, gru_Wx, gru_Wh, gru_bx, gru_bh, W5, b5):
  x_pad = jnp.pad(x, ((0, N_PAD - N), (0, 0)))
  src = edge_index[0]
  dst = edge_index[1]
  pad_idx = N + (jnp.arange(E_PAD - E, dtype=jnp.int32) % (N_PAD - N))
  srcp = jnp.concatenate([src, pad_idx])
  dstp = jnp.concatenate([dst, pad_idx])
  gru = (gru_Wx, gru_Wh, gru_bx, gru_bh)
  W5p = jnp.pad(W5, ((0, 0), (0, D - 1)))
  b5p = jnp.pad(b5, (0, D - 1))

  def sc(fn, z, elt, ert, H):
    cols = lambda m: [m[:, h] for h in range(H)]
    return fn(z, *cols(elt), *cols(ert), srcp, dstp)

  z, elt, ert = _tc_first(x_pad, W1, al1, ar1)
  acc, s = sc(_sc_edge4, z, elt, ert, 4)
  h1, z, elt, ert = _tc_t2(acc, _st(s), W2, al2, ar2)
  acc, s = sc(_sc_edge4, z, elt, ert, 4)
  h2, z, elt, ert = _tc_t3(acc, _st(s), h1, *gru, W3, al3, ar3)
  acc, s = sc(_sc_edge4, z, elt, ert, 4)
  h3, z, elt, ert = _tc_t4(acc, _st(s), h2, *gru, W4, al4, ar4)
  acc, s = sc(_sc_edge1, z, elt, ert, 1)
  (out_full,) = _tc_final(acc, _st(s), h3, *gru, W5p, b5p)
  return out_full[:N, :1]


# speedup vs baseline: 1.1879x; 1.0073x over previous
import functools

import jax
import jax.numpy as jnp
from jax import lax
from jax.experimental import pallas as pl
from jax.experimental.pallas import tpu as pltpu
from jax.experimental.pallas import tpu_sc as plsc

N = 10000
E = 320000
D = 128
N_PAD = 10240
E_PAD = 327680
NT = 16
NSC = 2
CHUNK = 128
TILE_E = E_PAD // (NSC * NT)
NCHUNK = TILE_E // CHUNK
NPIPE = NCHUNK // 2
ZROWS = 32
ROWS_T = N_PAD // NT
BN = 1024
GRID = N_PAD // BN


def _head_outputs(hcur, W_ref, al_ref, ar_ref, z_ref, elt_ref, ert_ref, Hn):
  for h in range(Hn):
    zh = jnp.dot(hcur, W_ref[h], preferred_element_type=jnp.float32)
    z_ref[h] = zh
    elt_ref[:, h:h + 1] = jnp.sum(zh * al_ref[h][None, :], axis=1,
                                  keepdims=True)
    ert_ref[:, h:h + 1] = jnp.sum(zh * ar_ref[h][None, :], axis=1,
                                  keepdims=True)


def _make_tc_first(H):
  def body(x_ref, W_ref, al_ref, ar_ref, z_ref, elt_ref, ert_ref):
    _head_outputs(x_ref[...], W_ref, al_ref, ar_ref, z_ref, elt_ref, ert_ref,
                  H)

  return pl.pallas_call(
      body,
      grid=(GRID,),
      in_specs=[
          pl.BlockSpec((BN, D), lambda i: (i, 0)),
          pl.BlockSpec((H, D, D), lambda i: (0, 0, 0)),
          pl.BlockSpec((H, D), lambda i: (0, 0)),
          pl.BlockSpec((H, D), lambda i: (0, 0)),
      ],
      out_specs=[
          pl.BlockSpec((H, BN, D), lambda i: (0, i, 0)),
          pl.BlockSpec((BN, H), lambda i: (i, 0)),
          pl.BlockSpec((BN, H), lambda i: (i, 0)),
      ],
      out_shape=[
          jax.ShapeDtypeStruct((H, N_PAD, D), jnp.float32),
          jax.ShapeDtypeStruct((N_PAD, H), jnp.float32),
          jax.ShapeDtypeStruct((N_PAD, H), jnp.float32),
      ],
  )


def _aggregate(acc_ref, st_ref, Hp):
  tot = None
  for h in range(Hp):
    zc = acc_ref[h, 0] + acc_ref[h, 1]
    sv = st_ref[:, 2 * h:2 * h + 1] + st_ref[:, 2 * h + 1:2 * h + 2]
    v = zc / (sv + 1e-9)
    tot = v if tot is None else tot + v
  a = tot * (1.0 / Hp)
  return jnp.where(a > 0, a, jnp.exp(jnp.minimum(a, 0.0)) - 1.0)


def _gru(a, hp, gwx_ref, gwh_ref, gbx_ref, gbh_ref):
  gx = jnp.dot(a, gwx_ref[...], preferred_element_type=jnp.float32)
  gx = gx + gbx_ref[...][None, :]
  gh = jnp.dot(hp, gwh_ref[...], preferred_element_type=jnp.float32)
  gh = gh + gbh_ref[...][None, :]
  r = jax.nn.sigmoid(gx[:, :D] + gh[:, :D])
  zt = jax.nn.sigmoid(gx[:, D:2 * D] + gh[:, D:2 * D])
  ng = jnp.tanh(gx[:, 2 * D:] + r * gh[:, 2 * D:])
  return (1.0 - zt) * ng + zt * hp


def _make_tc_trans(Hp, Hn, has_gru):
  if has_gru:
    def body(acc_ref, st_ref, hprev_ref, gwx_ref, gwh_ref, gbx_ref, gbh_ref,
             W_ref, al_ref, ar_ref, h_ref, z_ref, elt_ref, ert_ref):
      a = _aggregate(acc_ref, st_ref, Hp)
      hcur = _gru(a, hprev_ref[...], gwx_ref, gwh_ref, gbx_ref, gbh_ref)
      h_ref[...] = hcur
      _head_outputs(hcur, W_ref, al_ref, ar_ref, z_ref, elt_ref, ert_ref, Hn)

    extra_specs = [
        pl.BlockSpec((BN, D), lambda i: (i, 0)),
        pl.BlockSpec((D, 3 * D), lambda i: (0, 0)),
        pl.BlockSpec((D, 3 * D), lambda i: (0, 0)),
        pl.BlockSpec((3 * D,), lambda i: (0,)),
        pl.BlockSpec((3 * D,), lambda i: (0,)),
    ]
  else:
    def body(acc_ref, st_ref, W_ref, al_ref, ar_ref, h_ref, z_ref, elt_ref,
             ert_ref):
      a = _aggregate(acc_ref, st_ref, Hp)
      h_ref[...] = a
      _head_outputs(a, W_ref, al_ref, ar_ref, z_ref, elt_ref, ert_ref, Hn)

    extra_specs = []

  return pl.pallas_call(
      body,
      grid=(GRID,),
      in_specs=[
          pl.BlockSpec((Hp, NSC, BN, D), lambda i: (0, 0, i, 0)),
          pl.BlockSpec((BN, NSC * Hp), lambda i: (i, 0)),
      ] + extra_specs + [
          pl.BlockSpec((Hn, D, D), lambda i: (0, 0, 0)),
          pl.BlockSpec((Hn, D), lambda i: (0, 0)),
          pl.BlockSpec((Hn, D), lambda i: (0, 0)),
      ],
      out_specs=[
          pl.BlockSpec((BN, D), lambda i: (i, 0)),
          pl.BlockSpec((Hn, BN, D), lambda i: (0, i, 0)),
          pl.BlockSpec((BN, Hn), lambda i: (i, 0)),
          pl.BlockSpec((BN, Hn), lambda i: (i, 0)),
      ],
      out_shape=[
          jax.ShapeDtypeStruct((N_PAD, D), jnp.float32),
          jax.ShapeDtypeStruct((Hn, N_PAD, D), jnp.float32),
          jax.ShapeDtypeStruct((N_PAD, Hn), jnp.float32),
          jax.ShapeDtypeStruct((N_PAD, Hn), jnp.float32),
      ],
  )


def _make_tc_final():
  def body(acc_ref, st_ref, hprev_ref, gwx_ref, gwh_ref, gbx_ref, gbh_ref,
           W5_ref, b5_ref, out_ref):
    a = _aggregate(acc_ref, st_ref, 1)
    hcur = _gru(a, hprev_ref[...], gwx_ref, gwh_ref, gbx_ref, gbh_ref)
    y = jnp.dot(hcur, W5_ref[...], preferred_element_type=jnp.float32)
    out_ref[...] = jax.nn.sigmoid(y + b5_ref[...][None, :])

  return pl.pallas_call(
      body,
      grid=(GRID,),
      in_specs=[
          pl.BlockSpec((1, NSC, BN, D), lambda i: (0, 0, i, 0)),
          pl.BlockSpec((BN, NSC), lambda i: (i, 0)),
          pl.BlockSpec((BN, D), lambda i: (i, 0)),
          pl.BlockSpec((D, 3 * D), lambda i: (0, 0)),
          pl.BlockSpec((D, 3 * D), lambda i: (0, 0)),
          pl.BlockSpec((3 * D,), lambda i: (0,)),
          pl.BlockSpec((3 * D,), lambda i: (0,)),
          pl.BlockSpec((D, D), lambda i: (0, 0)),
          pl.BlockSpec((D,), lambda i: (0,)),
      ],
      out_specs=[pl.BlockSpec((BN, D), lambda i: (i, 0))],
      out_shape=[jax.ShapeDtypeStruct((N_PAD, D), jnp.float32)],
  )


def _make_sc_edge(H):
  mesh = plsc.VectorSubcoreMesh(core_axis_name="c", subcore_axis_name="s",
                                num_cores=NSC, num_subcores=NT)

  out_type = (
      jax.ShapeDtypeStruct((H, NSC, N_PAD, D), jnp.float32),
      jax.ShapeDtypeStruct((H, NSC, N_PAD), jnp.float32),
  )
  scratch = [
      pltpu.VMEM_SHARED((N_PAD, D), jnp.float32),
      pltpu.VMEM_SHARED((N_PAD,), jnp.float32),
      pltpu.VMEM_SHARED((N_PAD,), jnp.float32),
      pltpu.VMEM_SHARED((N_PAD,), jnp.float32),
      [pltpu.VMEM((CHUNK,), jnp.int32) for _ in range(2)],
      [pltpu.VMEM((CHUNK,), jnp.int32) for _ in range(2)],
      [pltpu.VMEM((CHUNK,), jnp.float32) for _ in range(2)],
      [pltpu.VMEM((CHUNK,), jnp.float32) for _ in range(2)],
      [pltpu.VMEM((CHUNK,), jnp.float32) for _ in range(2)],
      [pltpu.VMEM((CHUNK, D), jnp.float32) for _ in range(2)],
      pltpu.VMEM((ZROWS, D), jnp.float32),
      pltpu.VMEM((ROWS_T,), jnp.float32),
      [pltpu.SemaphoreType.DMA for _ in range(2)],
      [pltpu.SemaphoreType.DMA for _ in range(2)],
      [pltpu.SemaphoreType.DMA for _ in range(2)],
      [pltpu.SemaphoreType.DMA for _ in range(2)],
      [pltpu.SemaphoreType.DMA for _ in range(2)],
  ]

  @functools.partial(pl.kernel, out_type=out_type, mesh=mesh,
                     scratch_types=scratch)
  def k(*refs):
    (z_hbm, els, ers, src_hbm, dst_hbm, acc_out, s_out,
     acc_sp, s_sp, el_sp, er_sp, srcv, dstv, elv, erv, exv, rows, zbuf, zsv,
     gsem, ssem, esem, xsem, isem) = (refs[0], refs[1:1 + H],
                                      refs[1 + H:1 + 2 * H],
                                      *refs[1 + 2 * H:])
    c = lax.axis_index("c")
    t = lax.axis_index("s")
    rbase = t * ROWS_T
    ebase = c * (E_PAD // NSC) + t * TILE_E

    def zrow(i, carry):
      for q in range(D // 16):
        zbuf[i, pl.ds(q * 16, 16)] = jnp.zeros((16,), jnp.float32)
      return carry

    lax.fori_loop(0, ZROWS, zrow, 0)

    def zvec(i, carry):
      zsv[pl.ds(i * 16, 16)] = jnp.zeros((16,), jnp.float32)
      return carry

    lax.fori_loop(0, ROWS_T // 16, zvec, 0)

    for h in range(H):

      def load_idx(b, base):
        scp = pltpu.async_copy(src_hbm.at[pl.ds(base, CHUNK)], srcv[b],
                               isem[b])
        dcp = pltpu.async_copy(dst_hbm.at[pl.ds(base, CHUNK)], dstv[b],
                               isem[b])
        scp.wait()
        dcp.wait()
        pltpu.async_copy(el_sp.at[srcv[b]], elv[b], esem[b])
        pltpu.async_copy(er_sp.at[dstv[b]], erv[b], esem[b])
        pltpu.async_copy(z_hbm.at[h].at[srcv[b]], rows[b], gsem[b])

      def process(b):
        pltpu.make_async_copy(el_sp.at[srcv[b]], elv[b], esem[b]).wait()
        pltpu.make_async_copy(er_sp.at[dstv[b]], erv[b], esem[b]).wait()
        for q in range(CHUNK // 16):
          sl = pl.ds(q * 16, 16)
          e = elv[b][sl] + erv[b][sl]
          e = jnp.where(e >= 0.0, e, 0.2 * e)
          exv[b][sl] = jnp.exp(e)
        pltpu.async_copy(exv[b], s_sp.at[dstv[b]], xsem[b], add=True)
        pltpu.make_async_copy(z_hbm.at[h].at[srcv[b]], rows[b],
                              gsem[b]).wait()

        def row_body(g, rcarry):
          exvec = exv[b][pl.ds(g * 16, 16)]
          for r in range(16):
            i = g * 16 + r
            scv = jnp.full((16,), exvec[r], jnp.float32)
            for q in range(D // 16):
              sl = pl.ds(q * 16, 16)
              rows[b][i, sl] = rows[b][i, sl] * scv
          return rcarry

        lax.fori_loop(0, CHUNK // 16, row_body, 0)
        pltpu.async_copy(rows[b], acc_sp.at[dstv[b]], ssem[b], add=True)

      def drain_scatter(b):
        pltpu.make_async_copy(rows[b], acc_sp.at[dstv[b]], ssem[b]).wait()
        pltpu.make_async_copy(exv[b], s_sp.at[dstv[b]], xsem[b]).wait()

      pltpu.sync_copy(els[h].at[pl.ds(rbase, ROWS_T)],
                      el_sp.at[pl.ds(rbase, ROWS_T)])
      pltpu.sync_copy(ers[h].at[pl.ds(rbase, ROWS_T)],
                      er_sp.at[pl.ds(rbase, ROWS_T)])
      for jj in range(ROWS_T // ZROWS):
        pltpu.sync_copy(zbuf, acc_sp.at[pl.ds(rbase + jj * ZROWS, ZROWS)])
      pltpu.sync_copy(zsv, s_sp.at[pl.ds(rbase, ROWS_T)])
      plsc.subcore_barrier()

      load_idx(0, ebase)

      def pipe_body(jj, carry):
        base_a = ebase + jj * (2 * CHUNK)

        @pl.when(jj > 0)
        def _():
          drain_scatter(1)

        load_idx(1, base_a + CHUNK)
        process(0)
        process(1)

        @pl.when(jj < NPIPE - 1)
        def _():
          drain_scatter(0)
          load_idx(0, base_a + 2 * CHUNK)

        return carry

      lax.fori_loop(0, NPIPE, pipe_body, 0)
      drain_scatter(0)
      drain_scatter(1)
      plsc.subcore_barrier()

      pltpu.sync_copy(acc_sp.at[pl.ds(rbase, ROWS_T)],
                      acc_out.at[h, c, pl.ds(rbase, ROWS_T)])
      pltpu.sync_copy(s_sp.at[pl.ds(rbase, ROWS_T)],
                      s_out.at[h, c, pl.ds(rbase, ROWS_T)])

  return k


_tc_first = _make_tc_first(4)
_tc_t2 = _make_tc_trans(4, 4, False)
_tc_t3 = _make_tc_trans(4, 4, True)
_tc_t4 = _make_tc_trans(4, 1, True)
_tc_final = _make_tc_final()
_sc_edge4 = _make_sc_edge(4)
_sc_edge1 = _make_sc_edge(1)


def _st(s):
  return jnp.transpose(s, (2, 0, 1)).reshape(N_PAD, -1)


def kernel(x, edge_index, W1, al1, ar1, W2, al2, ar2, W3, al3, ar3,
           W4, al4, ar4, gru_Wx, gru_Wh, gru_bx, gru_bh, W5, b5):
  x_pad = jnp.pad(x, ((0, N_PAD - N), (0, 0)))
  src = edge_index[0]
  dst = edge_index[1]
  pad_idx = N + (jnp.arange(E_PAD - E, dtype=jnp.int32) % (N_PAD - N))
  srcp = jnp.concatenate([src, pad_idx])
  dstp = jnp.concatenate([dst, pad_idx])
  gru = (gru_Wx, gru_Wh, gru_bx, gru_bh)
  W5p = jnp.pad(W5, ((0, 0), (0, D - 1)))
  b5p = jnp.pad(b5, (0, D - 1))

  def sc(fn, z, elt, ert, H):
    cols = lambda m: [m[:, h] for h in range(H)]
    return fn(z, *cols(elt), *cols(ert), srcp, dstp)

  z, elt, ert = _tc_first(x_pad, W1, al1, ar1)
  acc, s = sc(_sc_edge4, z, elt, ert, 4)
  h1, z, elt, ert = _tc_t2(acc, _st(s), W2, al2, ar2)
  acc, s = sc(_sc_edge4, z, elt, ert, 4)
  h2, z, elt, ert = _tc_t3(acc, _st(s), h1, *gru, W3, al3, ar3)
  acc, s = sc(_sc_edge4, z, elt, ert, 4)
  h3, z, elt, ert = _tc_t4(acc, _st(s), h2, *gru, W4, al4, ar4)
  acc, s = sc(_sc_edge1, z, elt, ert, 1)
  (out_full,) = _tc_final(acc, _st(s), h3, *gru, W5p, b5p)
  return out_full[:N, :1]
